# baseline (speedup 1.0000x reference)
import functools

import jax
import jax.numpy as jnp
import numpy as np
from jax import lax
from jax.experimental import pallas as pl
from jax.experimental.pallas import tpu as pltpu

_B, _Q, _C, _G = 4, 300, 2, 8
_NS = 100
_NBP = 17
_J, _V = 137, 10475
_JF = _J * 3
_VF = _V * 3

_OFF_BOX = 0
_OFF_KP = 4
_OFF_POSE = 55
_OFF_BETA = 214
_OFF_EXPR = 224
_OFF_CAM = 234
_OFF_KP3D = 237
_F = 648


def _build_m_d3():
    m = np.eye(_F, dtype=np.float32)
    m[_OFF_BOX:_OFF_BOX + 4, _OFF_BOX:_OFF_BOX + 4] = 0.0
    m[_OFF_BOX + 0, _OFF_BOX + 0] = 1.0
    m[_OFF_BOX + 2, _OFF_BOX + 0] = -0.5
    m[_OFF_BOX + 1, _OFF_BOX + 1] = 1.0
    m[_OFF_BOX + 3, _OFF_BOX + 1] = -0.5
    m[_OFF_BOX + 0, _OFF_BOX + 2] = 1.0
    m[_OFF_BOX + 2, _OFF_BOX + 2] = 0.5
    m[_OFF_BOX + 1, _OFF_BOX + 3] = 1.0
    m[_OFF_BOX + 3, _OFF_BOX + 3] = 0.5
    m[_OFF_KP:_OFF_KP + 51, _OFF_KP:_OFF_KP + 51] = 0.0
    for k in range(_NBP):
        m[_OFF_KP + 2 * k, _OFF_KP + 3 * k] = 1.0
        m[_OFF_KP + 2 * k + 1, _OFF_KP + 3 * k + 1] = 1.0
        m[_OFF_KP + 2 * _NBP + k, _OFF_KP + 3 * k + 2] = 1.0
    d3 = np.zeros((3, _F), dtype=np.float32)
    d3[2, :] = 1.0
    for j, r in ((0, 0), (1, 1), (2, 0), (3, 1)):
        d3[2, _OFF_BOX + j] = 0.0
        d3[r, _OFF_BOX + j] = 1.0
    for k in range(_NBP):
        d3[2, _OFF_KP + 3 * k] = 0.0
        d3[0, _OFF_KP + 3 * k] = 1.0
        d3[2, _OFF_KP + 3 * k + 1] = 0.0
        d3[1, _OFF_KP + 3 * k + 1] = 1.0
    return m, d3


_M_CONST, _D3_CONST = _build_m_d3()
_HI = jax.lax.Precision.HIGHEST


def _select_kernel(prob_ref, feat_ref, kp3d_ref, gt_ref, valid_ref, ts_ref,
                   m_ref, d3_ref,
                   scores_ref, labels_ref, fqflat_ref, big_ref):
    f32 = jnp.float32
    i32 = jnp.int32
    iota6 = lax.broadcasted_iota(i32, (_B, _Q * _C), 1)
    iota100 = lax.broadcasted_iota(i32, (_B, _NS), 1)

    def topk_body(p, carry):
        cur, vals, idxs = carry
        m = jnp.max(cur, axis=1, keepdims=True)
        amax = jnp.min(jnp.where(cur == m, iota6, _Q * _C), axis=1,
                       keepdims=True)
        at_p = iota100 == p
        vals = jnp.where(at_p, m, vals)
        idxs = jnp.where(at_p, amax, idxs)
        return (jnp.where(iota6 == amax, -1.0, cur), vals, idxs)

    _, vals, idxs = lax.fori_loop(
        0, _NS, topk_body,
        (prob_ref[...], jnp.zeros((_B, _NS), f32),
         jnp.zeros((_B, _NS), i32)))
    tb = idxs // _C
    lab = idxs % _C

    oh1 = (lax.broadcasted_iota(i32, (_B, _NS, _Q), 2)
           == tb[:, :, None]).astype(f32)
    selk = lax.dot_general(oh1, kp3d_ref[...],
                           (((2,), (1,)), ((0,), (0,))),
                           precision=_HI)

    costs = []
    for g in range(_G):
        gt_g = gt_ref[:, g, :][:, None, :]
        v_g = valid_ref[:, g, :][:, None, :]
        costs.append(jnp.sum(jnp.abs((selk - gt_g) * v_g), axis=2))
    cost = jnp.stack(costs, axis=1)

    used = jnp.zeros((_B, _NS), jnp.bool_)
    fq_cols, sc_cols, lb_cols = [], [], []
    for g in range(_G):
        c = jnp.where(used, jnp.inf, cost[:, g, :])
        m = jnp.min(c, axis=1, keepdims=True)
        idx_g = jnp.min(jnp.where(c == m, iota100, _NS), axis=1,
                        keepdims=True)
        sel = iota100 == idx_g
        used = jnp.logical_or(used, sel)
        sc_cols.append(jnp.sum(sel.astype(f32) * vals, axis=1,
                               keepdims=True))
        lb_cols.append(jnp.sum(jnp.where(sel, lab, 0), axis=1,
                               keepdims=True))
        fq_cols.append(jnp.sum(jnp.where(sel, tb, 0), axis=1, keepdims=True))

    fq = jnp.concatenate(fq_cols, axis=1)
    boff = lax.broadcasted_iota(i32, (_B, _G), 0) * _Q
    scores_ref[...] = jnp.concatenate(sc_cols, axis=1)
    labels_ref[...] = jnp.concatenate(lb_cols, axis=1)
    fqflat_ref[...] = fq + boff

    oh2 = (lax.broadcasted_iota(i32, (_B, _G, _Q), 2)
           == fq[:, :, None]).astype(f32)
    sel8 = lax.dot_general(oh2, feat_ref[...],
                           (((2,), (1,)), ((0,), (0,))),
                           precision=_HI)
    t8 = lax.dot_general(sel8, m_ref[...],
                         (((2,), (0,)), ((), ())),
                         precision=_HI)
    w = ts_ref[:, 1:2]
    h = ts_ref[:, 0:1]
    d = w * d3_ref[0:1, :] + h * d3_ref[1:2, :] + d3_ref[2:3, :]
    big_ref[...] = t8 * d[:, None, :]


def _gather_rows_kernel(idx_ref, rows_ref, out_ref):
    out_ref[...] = rows_ref[...]


@jax.jit
def kernel(pred_logits, pred_boxes, pred_keypoints, pred_smpl_fullpose,
           pred_smpl_beta, pred_smpl_expr, pred_smpl_cam, pred_smpl_kp3d,
           pred_smpl_verts, target_sizes, joint_cam, joint_valid,
           body_bbox_center, body_bbox_size):
    del body_bbox_center, body_bbox_size
    f32 = jnp.float32
    prob = jax.nn.sigmoid(pred_logits).reshape(_B, _Q * _C)
    kp3d_flat = pred_smpl_kp3d.reshape(_B, _Q, _JF)
    feat = jnp.concatenate(
        [pred_boxes, pred_keypoints, pred_smpl_fullpose, pred_smpl_beta,
         pred_smpl_expr, pred_smpl_cam, kp3d_flat], axis=-1)
    gt = joint_cam.reshape(_B, _G, _JF)
    validx = jnp.broadcast_to(joint_valid, (_B, _G, _J, 3)).reshape(
        _B, _G, _JF)

    scores, labels, fqflat, big = pl.pallas_call(
        _select_kernel,
        out_shape=[
            jax.ShapeDtypeStruct((_B, _G), f32),
            jax.ShapeDtypeStruct((_B, _G), jnp.int32),
            jax.ShapeDtypeStruct((_B, _G), jnp.int32),
            jax.ShapeDtypeStruct((_B, _G, _F), f32),
        ],
    )(prob, feat, kp3d_flat, gt, validx, target_sizes,
      jnp.asarray(_M_CONST), jnp.asarray(_D3_CONST))

    verts2 = pred_smpl_verts.reshape(_B * _Q, 1, _VF)
    verts_sel = pl.pallas_call(
        _gather_rows_kernel,
        grid_spec=pltpu.PrefetchScalarGridSpec(
            num_scalar_prefetch=1,
            grid=(_B * _G,),
            in_specs=[pl.BlockSpec((1, 1, _VF),
                                   lambda i, idx: (idx[i], 0, 0))],
            out_specs=pl.BlockSpec((1, 1, _VF), lambda i, idx: (i, 0, 0)),
        ),
        out_shape=jax.ShapeDtypeStruct((_B * _G, 1, _VF), f32),
    )(fqflat.reshape(_B * _G), verts2)

    boxes_sc = big[..., _OFF_BOX:_OFF_BOX + 4]
    kr = big[..., _OFF_KP:_OFF_KP + 51]
    pose = big[..., _OFF_POSE:_OFF_POSE + 159]
    beta = big[..., _OFF_BETA:_OFF_BETA + 10]
    expr = big[..., _OFF_EXPR:_OFF_EXPR + 10]
    cam = big[..., _OFF_CAM:_OFF_CAM + 3]
    kp3d = big[..., _OFF_KP3D:].reshape(_B, _G, _J, 3)
    verts = verts_sel.reshape(_B, _G, _V, 3)
    return (scores, labels, boxes_sc, kr, pose, beta, expr, cam, kp3d, verts)


# speedup vs baseline: 11.0779x; 11.0779x over previous
import functools

import jax
import jax.numpy as jnp
import numpy as np
from jax import lax
from jax.experimental import pallas as pl
from jax.experimental.pallas import tpu as pltpu

_B, _Q, _C, _G = 4, 300, 2, 8
_NS = 100
_NBP = 17
_J, _V = 137, 10475
_JF = _J * 3
_VF = _V * 3

_OFF_BOX = 0
_OFF_KP = 4
_OFF_POSE = 55
_OFF_BETA = 214
_OFF_EXPR = 224
_OFF_CAM = 234
_OFF_KP3D = 237
_F = 648


def _build_m_d3():
    m = np.eye(_F, dtype=np.float32)
    m[_OFF_BOX:_OFF_BOX + 4, _OFF_BOX:_OFF_BOX + 4] = 0.0
    m[_OFF_BOX + 0, _OFF_BOX + 0] = 1.0
    m[_OFF_BOX + 2, _OFF_BOX + 0] = -0.5
    m[_OFF_BOX + 1, _OFF_BOX + 1] = 1.0
    m[_OFF_BOX + 3, _OFF_BOX + 1] = -0.5
    m[_OFF_BOX + 0, _OFF_BOX + 2] = 1.0
    m[_OFF_BOX + 2, _OFF_BOX + 2] = 0.5
    m[_OFF_BOX + 1, _OFF_BOX + 3] = 1.0
    m[_OFF_BOX + 3, _OFF_BOX + 3] = 0.5
    m[_OFF_KP:_OFF_KP + 51, _OFF_KP:_OFF_KP + 51] = 0.0
    for k in range(_NBP):
        m[_OFF_KP + 2 * k, _OFF_KP + 3 * k] = 1.0
        m[_OFF_KP + 2 * k + 1, _OFF_KP + 3 * k + 1] = 1.0
        m[_OFF_KP + 2 * _NBP + k, _OFF_KP + 3 * k + 2] = 1.0
    d3 = np.zeros((3, _F), dtype=np.float32)
    d3[2, :] = 1.0
    for j, r in ((0, 0), (1, 1), (2, 0), (3, 1)):
        d3[2, _OFF_BOX + j] = 0.0
        d3[r, _OFF_BOX + j] = 1.0
    for k in range(_NBP):
        d3[2, _OFF_KP + 3 * k] = 0.0
        d3[0, _OFF_KP + 3 * k] = 1.0
        d3[2, _OFF_KP + 3 * k + 1] = 0.0
        d3[1, _OFF_KP + 3 * k + 1] = 1.0
    return m, d3


_M_CONST, _D3_CONST = _build_m_d3()
_HI = jax.lax.Precision.HIGHEST


def _select_kernel(prob_ref, feat_ref, kp3d_ref, gt_ref, valid_ref, ts_ref,
                   m_ref, d3_ref,
                   scores_ref, labels_ref, fqflat_ref, big_ref):
    f32 = jnp.float32
    i32 = jnp.int32
    iota6 = lax.broadcasted_iota(i32, (_B, _Q * _C), 1)
    iota100 = lax.broadcasted_iota(i32, (_B, _NS), 1)

    def topk_body(p, carry):
        cur, vals, idxs = carry
        m = jnp.max(cur, axis=1, keepdims=True)
        amax = jnp.min(jnp.where(cur == m, iota6, _Q * _C), axis=1,
                       keepdims=True)
        at_p = iota100 == p
        vals = jnp.where(at_p, m, vals)
        idxs = jnp.where(at_p, amax, idxs)
        return (jnp.where(iota6 == amax, -1.0, cur), vals, idxs)

    _, vals, idxs = lax.fori_loop(
        0, _NS, topk_body,
        (prob_ref[...], jnp.zeros((_B, _NS), f32),
         jnp.zeros((_B, _NS), i32)))
    tb = idxs // _C
    lab = idxs % _C

    oh1 = (lax.broadcasted_iota(i32, (_B, _NS, _Q), 2)
           == tb[:, :, None]).astype(f32)
    selk = lax.dot_general(oh1, kp3d_ref[...],
                           (((2,), (1,)), ((0,), (0,))),
                           precision=_HI)

    costs = []
    for g in range(_G):
        gt_g = gt_ref[:, g, :][:, None, :]
        v_g = valid_ref[:, g, :][:, None, :]
        costs.append(jnp.sum(jnp.abs((selk - gt_g) * v_g), axis=2))
    cost = jnp.stack(costs, axis=1)

    used = jnp.zeros((_B, _NS), jnp.bool_)
    fq_cols, sc_cols, lb_cols = [], [], []
    for g in range(_G):
        c = jnp.where(used, jnp.inf, cost[:, g, :])
        m = jnp.min(c, axis=1, keepdims=True)
        idx_g = jnp.min(jnp.where(c == m, iota100, _NS), axis=1,
                        keepdims=True)
        sel = iota100 == idx_g
        used = jnp.logical_or(used, sel)
        sc_cols.append(jnp.sum(sel.astype(f32) * vals, axis=1,
                               keepdims=True))
        lb_cols.append(jnp.sum(jnp.where(sel, lab, 0), axis=1,
                               keepdims=True))
        fq_cols.append(jnp.sum(jnp.where(sel, tb, 0), axis=1, keepdims=True))

    fq = jnp.concatenate(fq_cols, axis=1)
    scores_ref[...] = jnp.concatenate(sc_cols, axis=1)
    labels_ref[...] = jnp.concatenate(lb_cols, axis=1)
    fqflat_ref[...] = fq

    oh2 = (lax.broadcasted_iota(i32, (_B, _G, _Q), 2)
           == fq[:, :, None]).astype(f32)
    sel8 = lax.dot_general(oh2, feat_ref[...],
                           (((2,), (1,)), ((0,), (0,))),
                           precision=_HI)
    t8 = lax.dot_general(sel8, m_ref[...],
                         (((2,), (0,)), ((), ())),
                         precision=_HI)
    w = ts_ref[:, 1:2]
    h = ts_ref[:, 0:1]
    d = w * d3_ref[0:1, :] + h * d3_ref[1:2, :] + d3_ref[2:3, :]
    big_ref[...] = t8 * d[:, None, :]


def _gather_rows_kernel(idx_ref, rows_ref, out_ref):
    b = pl.program_id(0) // _G
    x = rows_ref[0, 0]
    mask = lax.broadcasted_iota(jnp.int32, (_B, _V), 0) == b
    out_ref[0, 0] = jnp.sum(jnp.where(mask, x, 0.0), axis=0, keepdims=True)


@jax.jit
def kernel(pred_logits, pred_boxes, pred_keypoints, pred_smpl_fullpose,
           pred_smpl_beta, pred_smpl_expr, pred_smpl_cam, pred_smpl_kp3d,
           pred_smpl_verts, target_sizes, joint_cam, joint_valid,
           body_bbox_center, body_bbox_size):
    del body_bbox_center, body_bbox_size
    f32 = jnp.float32
    prob = jax.nn.sigmoid(pred_logits).reshape(_B, _Q * _C)
    kp3d_flat = pred_smpl_kp3d.reshape(_B, _Q, _JF)
    feat = jnp.concatenate(
        [pred_boxes, pred_keypoints, pred_smpl_fullpose, pred_smpl_beta,
         pred_smpl_expr, pred_smpl_cam, kp3d_flat], axis=-1)
    gt = joint_cam.reshape(_B, _G, _JF)
    validx = jnp.broadcast_to(joint_valid, (_B, _G, _J, 3)).reshape(
        _B, _G, _JF)

    scores, labels, fq8, big = pl.pallas_call(
        _select_kernel,
        out_shape=[
            jax.ShapeDtypeStruct((_B, _G), f32),
            jax.ShapeDtypeStruct((_B, _G), jnp.int32),
            jax.ShapeDtypeStruct((_B, _G), jnp.int32),
            jax.ShapeDtypeStruct((_B, _G, _F), f32),
        ],
    )(prob, feat, kp3d_flat, gt, validx, target_sizes,
      jnp.asarray(_M_CONST), jnp.asarray(_D3_CONST))

    verts_t = jnp.transpose(pred_smpl_verts, (1, 3, 0, 2))
    verts_sel = pl.pallas_call(
        _gather_rows_kernel,
        grid_spec=pltpu.PrefetchScalarGridSpec(
            num_scalar_prefetch=1,
            grid=(_B * _G, 3),
            in_specs=[pl.BlockSpec((1, 1, _B, _V),
                                   lambda i, c, idx: (idx[i], c, 0, 0))],
            out_specs=pl.BlockSpec((1, 1, 1, _V),
                                   lambda i, c, idx: (i, c, 0, 0)),
        ),
        out_shape=jax.ShapeDtypeStruct((_B * _G, 3, 1, _V), f32),
    )(fq8.reshape(_B * _G), verts_t)

    boxes_sc = big[..., _OFF_BOX:_OFF_BOX + 4]
    kr = big[..., _OFF_KP:_OFF_KP + 51]
    pose = big[..., _OFF_POSE:_OFF_POSE + 159]
    beta = big[..., _OFF_BETA:_OFF_BETA + 10]
    expr = big[..., _OFF_EXPR:_OFF_EXPR + 10]
    cam = big[..., _OFF_CAM:_OFF_CAM + 3]
    kp3d = big[..., _OFF_KP3D:].reshape(_B, _G, _J, 3)
    verts = jnp.transpose(verts_sel.reshape(_B, _G, 3, _V),
                          (0, 1, 3, 2))
    return (scores, labels, boxes_sc, kr, pose, beta, expr, cam, kp3d, verts)


# speedup vs baseline: 13.2355x; 1.1948x over previous
import functools

import jax
import jax.numpy as jnp
import numpy as np
from jax import lax
from jax.experimental import pallas as pl
from jax.experimental.pallas import tpu as pltpu

_B, _Q, _C, _G = 4, 300, 2, 8
_NS = 100
_NBP = 17
_J, _V = 137, 10475
_JF = _J * 3
_VF = _V * 3
_QC = _Q * _C

_OFF_BOX = 0
_OFF_KP = 4
_OFF_POSE = 55
_OFF_BETA = 214
_OFF_EXPR = 224
_OFF_CAM = 234
_OFF_KP3D = 237
_F = 648


def _build_m_d3():
    m = np.eye(_F, dtype=np.float32)
    m[_OFF_BOX:_OFF_BOX + 4, _OFF_BOX:_OFF_BOX + 4] = 0.0
    m[_OFF_BOX + 0, _OFF_BOX + 0] = 1.0
    m[_OFF_BOX + 2, _OFF_BOX + 0] = -0.5
    m[_OFF_BOX + 1, _OFF_BOX + 1] = 1.0
    m[_OFF_BOX + 3, _OFF_BOX + 1] = -0.5
    m[_OFF_BOX + 0, _OFF_BOX + 2] = 1.0
    m[_OFF_BOX + 2, _OFF_BOX + 2] = 0.5
    m[_OFF_BOX + 1, _OFF_BOX + 3] = 1.0
    m[_OFF_BOX + 3, _OFF_BOX + 3] = 0.5
    m[_OFF_KP:_OFF_KP + 51, _OFF_KP:_OFF_KP + 51] = 0.0
    for k in range(_NBP):
        m[_OFF_KP + 2 * k, _OFF_KP + 3 * k] = 1.0
        m[_OFF_KP + 2 * k + 1, _OFF_KP + 3 * k + 1] = 1.0
        m[_OFF_KP + 2 * _NBP + k, _OFF_KP + 3 * k + 2] = 1.0
    d3 = np.zeros((3, _F), dtype=np.float32)
    d3[2, :] = 1.0
    for j, r in ((0, 0), (1, 1), (2, 0), (3, 1)):
        d3[2, _OFF_BOX + j] = 0.0
        d3[r, _OFF_BOX + j] = 1.0
    for k in range(_NBP):
        d3[2, _OFF_KP + 3 * k] = 0.0
        d3[0, _OFF_KP + 3 * k] = 1.0
        d3[2, _OFF_KP + 3 * k + 1] = 0.0
        d3[1, _OFF_KP + 3 * k + 1] = 1.0
    return m, d3


_M_CONST, _D3_CONST = _build_m_d3()
_HI = jax.lax.Precision.HIGHEST


def _select_kernel(prob_ref, feat_ref, kp3d_ref, gt_ref, valid_ref, ts_ref,
                   m_ref, d3_ref,
                   scores_ref, labels_ref, fqflat_ref, big_ref):
    f32 = jnp.float32
    i32 = jnp.int32
    iota100 = lax.broadcasted_iota(i32, (_B, _NS), 1)

    p = prob_ref[...]
    pi = p[:, :, None]
    pj = p[:, None, :]
    jlti = (lax.broadcasted_iota(i32, (_B, _QC, _QC), 2)
            < lax.broadcasted_iota(i32, (_B, _QC, _QC), 1))
    beats = jnp.logical_or(pj > pi, jnp.logical_and(pj == pi, jlti))
    rank = jnp.sum(beats.astype(i32), axis=2)
    ohr = (rank[:, :, None]
           == lax.broadcasted_iota(i32, (_B, _QC, _NS), 2)).astype(f32)
    vals = lax.dot_general(p[:, None, :], ohr,
                           (((2,), (1,)), ((0,), (0,))),
                           precision=_HI)[:, 0, :]
    idxf = lax.broadcasted_iota(i32, (_B, _QC), 1).astype(f32)
    idxs = lax.dot_general(idxf[:, None, :], ohr,
                           (((2,), (1,)), ((0,), (0,))),
                           precision=_HI)[:, 0, :].astype(i32)
    tb = idxs // _C
    lab = idxs % _C

    oh1 = (lax.broadcasted_iota(i32, (_B, _NS, _Q), 2)
           == tb[:, :, None]).astype(f32)
    selk = lax.dot_general(oh1, kp3d_ref[...],
                           (((2,), (1,)), ((0,), (0,))),
                           precision=_HI)

    costs = []
    for g in range(_G):
        gt_g = gt_ref[:, g, :][:, None, :]
        v_g = valid_ref[:, g, :][:, None, :]
        costs.append(jnp.sum(jnp.abs((selk - gt_g) * v_g), axis=2))
    cost = jnp.stack(costs, axis=1)

    used = jnp.zeros((_B, _NS), jnp.bool_)
    fq_cols, sc_cols, lb_cols = [], [], []
    for g in range(_G):
        c = jnp.where(used, jnp.inf, cost[:, g, :])
        m = jnp.min(c, axis=1, keepdims=True)
        idx_g = jnp.min(jnp.where(c == m, iota100, _NS), axis=1,
                        keepdims=True)
        sel = iota100 == idx_g
        used = jnp.logical_or(used, sel)
        sc_cols.append(jnp.sum(sel.astype(f32) * vals, axis=1,
                               keepdims=True))
        lb_cols.append(jnp.sum(jnp.where(sel, lab, 0), axis=1,
                               keepdims=True))
        fq_cols.append(jnp.sum(jnp.where(sel, tb, 0), axis=1, keepdims=True))

    fq = jnp.concatenate(fq_cols, axis=1)
    scores_ref[...] = jnp.concatenate(sc_cols, axis=1)
    labels_ref[...] = jnp.concatenate(lb_cols, axis=1)
    fqflat_ref[...] = fq

    oh2 = (lax.broadcasted_iota(i32, (_B, _G, _Q), 2)
           == fq[:, :, None]).astype(f32)
    sel8 = lax.dot_general(oh2, feat_ref[...],
                           (((2,), (1,)), ((0,), (0,))),
                           precision=_HI)
    t8 = lax.dot_general(sel8, m_ref[...],
                         (((2,), (0,)), ((), ())),
                         precision=_HI)
    w = ts_ref[:, 1:2]
    h = ts_ref[:, 0:1]
    d = w * d3_ref[0:1, :] + h * d3_ref[1:2, :] + d3_ref[2:3, :]
    big_ref[...] = t8 * d[:, None, :]


def _gather_rows_kernel(idx_ref, rows_ref, out_ref):
    b = pl.program_id(0) // _G
    x = rows_ref[0, 0]
    mask = lax.broadcasted_iota(jnp.int32, (_B, _V), 0) == b
    out_ref[0, 0] = jnp.sum(jnp.where(mask, x, 0.0), axis=0, keepdims=True)


@jax.jit
def kernel(pred_logits, pred_boxes, pred_keypoints, pred_smpl_fullpose,
           pred_smpl_beta, pred_smpl_expr, pred_smpl_cam, pred_smpl_kp3d,
           pred_smpl_verts, target_sizes, joint_cam, joint_valid,
           body_bbox_center, body_bbox_size):
    del body_bbox_center, body_bbox_size
    f32 = jnp.float32
    prob = jax.nn.sigmoid(pred_logits).reshape(_B, _Q * _C)
    kp3d_flat = pred_smpl_kp3d.reshape(_B, _Q, _JF)
    feat = jnp.concatenate(
        [pred_boxes, pred_keypoints, pred_smpl_fullpose, pred_smpl_beta,
         pred_smpl_expr, pred_smpl_cam, kp3d_flat], axis=-1)
    gt = joint_cam.reshape(_B, _G, _JF)
    validx = jnp.broadcast_to(joint_valid, (_B, _G, _J, 3)).reshape(
        _B, _G, _JF)

    scores, labels, fq8, big = pl.pallas_call(
        _select_kernel,
        out_shape=[
            jax.ShapeDtypeStruct((_B, _G), f32),
            jax.ShapeDtypeStruct((_B, _G), jnp.int32),
            jax.ShapeDtypeStruct((_B, _G), jnp.int32),
            jax.ShapeDtypeStruct((_B, _G, _F), f32),
        ],
    )(prob, feat, kp3d_flat, gt, validx, target_sizes,
      jnp.asarray(_M_CONST), jnp.asarray(_D3_CONST))

    verts_t = jnp.transpose(pred_smpl_verts, (1, 3, 0, 2))
    verts_sel = pl.pallas_call(
        _gather_rows_kernel,
        grid_spec=pltpu.PrefetchScalarGridSpec(
            num_scalar_prefetch=1,
            grid=(_B * _G, 3),
            in_specs=[pl.BlockSpec((1, 1, _B, _V),
                                   lambda i, c, idx: (idx[i], c, 0, 0))],
            out_specs=pl.BlockSpec((1, 1, 1, _V),
                                   lambda i, c, idx: (i, c, 0, 0)),
        ),
        out_shape=jax.ShapeDtypeStruct((_B * _G, 3, 1, _V), f32),
    )(fq8.reshape(_B * _G), verts_t)

    boxes_sc = big[..., _OFF_BOX:_OFF_BOX + 4]
    kr = big[..., _OFF_KP:_OFF_KP + 51]
    pose = big[..., _OFF_POSE:_OFF_POSE + 159]
    beta = big[..., _OFF_BETA:_OFF_BETA + 10]
    expr = big[..., _OFF_EXPR:_OFF_EXPR + 10]
    cam = big[..., _OFF_CAM:_OFF_CAM + 3]
    kp3d = big[..., _OFF_KP3D:].reshape(_B, _G, _J, 3)
    verts = jnp.transpose(verts_sel.reshape(_B, _G, 3, _V),
                          (0, 1, 3, 2))
    return (scores, labels, boxes_sc, kr, pose, beta, expr, cam, kp3d, verts)


# speedup vs baseline: 18.2401x; 1.3781x over previous
import functools

import jax
import jax.numpy as jnp
import numpy as np
from jax import lax
from jax.experimental import pallas as pl
from jax.experimental.pallas import tpu as pltpu

_B, _Q, _C, _G = 4, 300, 2, 8
_NS = 100
_NBP = 17
_J, _V = 137, 10475
_JF = _J * 3
_VF = _V * 3
_QC = _Q * _C

_OFF_BOX = 0
_OFF_KP = 4
_OFF_POSE = 55
_OFF_BETA = 214
_OFF_EXPR = 224
_OFF_CAM = 234
_OFF_KP3D = 237
_F = 648


def _build_m_d3():
    m = np.eye(_F, dtype=np.float32)
    m[_OFF_BOX:_OFF_BOX + 4, _OFF_BOX:_OFF_BOX + 4] = 0.0
    m[_OFF_BOX + 0, _OFF_BOX + 0] = 1.0
    m[_OFF_BOX + 2, _OFF_BOX + 0] = -0.5
    m[_OFF_BOX + 1, _OFF_BOX + 1] = 1.0
    m[_OFF_BOX + 3, _OFF_BOX + 1] = -0.5
    m[_OFF_BOX + 0, _OFF_BOX + 2] = 1.0
    m[_OFF_BOX + 2, _OFF_BOX + 2] = 0.5
    m[_OFF_BOX + 1, _OFF_BOX + 3] = 1.0
    m[_OFF_BOX + 3, _OFF_BOX + 3] = 0.5
    m[_OFF_KP:_OFF_KP + 51, _OFF_KP:_OFF_KP + 51] = 0.0
    for k in range(_NBP):
        m[_OFF_KP + 2 * k, _OFF_KP + 3 * k] = 1.0
        m[_OFF_KP + 2 * k + 1, _OFF_KP + 3 * k + 1] = 1.0
        m[_OFF_KP + 2 * _NBP + k, _OFF_KP + 3 * k + 2] = 1.0
    d3 = np.zeros((3, _F), dtype=np.float32)
    d3[2, :] = 1.0
    for j, r in ((0, 0), (1, 1), (2, 0), (3, 1)):
        d3[2, _OFF_BOX + j] = 0.0
        d3[r, _OFF_BOX + j] = 1.0
    for k in range(_NBP):
        d3[2, _OFF_KP + 3 * k] = 0.0
        d3[0, _OFF_KP + 3 * k] = 1.0
        d3[2, _OFF_KP + 3 * k + 1] = 0.0
        d3[1, _OFF_KP + 3 * k + 1] = 1.0
    return m, d3


_M_CONST, _D3_CONST = _build_m_d3()
_HI = jax.lax.Precision.HIGHEST


def _select_kernel(prob_ref, feat_ref, kp3d_ref, gt_ref, valid_ref, ts_ref,
                   m_ref, d3_ref,
                   scores_ref, labels_ref, fqflat_ref, big_ref):
    f32 = jnp.float32
    i32 = jnp.int32
    iota100 = lax.broadcasted_iota(i32, (_B, _NS), 1)

    p = prob_ref[...]
    pi = p[:, :, None]
    pj = p[:, None, :]
    jlti = (lax.broadcasted_iota(i32, (_B, _QC, _QC), 2)
            < lax.broadcasted_iota(i32, (_B, _QC, _QC), 1))
    beats = jnp.logical_or(pj > pi, jnp.logical_and(pj == pi, jlti))
    rank = jnp.sum(beats.astype(i32), axis=2)
    ohr = (rank[:, :, None]
           == lax.broadcasted_iota(i32, (_B, _QC, _NS), 2)).astype(f32)
    vals = lax.dot_general(p[:, None, :], ohr,
                           (((2,), (1,)), ((0,), (0,))),
                           precision=_HI)[:, 0, :]
    idxf = lax.broadcasted_iota(i32, (_B, _QC), 1).astype(f32)
    idxs = lax.dot_general(idxf[:, None, :], ohr,
                           (((2,), (1,)), ((0,), (0,))),
                           precision=_HI)[:, 0, :].astype(i32)
    tb = idxs // _C
    lab = idxs % _C

    oh1 = (lax.broadcasted_iota(i32, (_B, _NS, _Q), 2)
           == tb[:, :, None]).astype(f32)
    selk = lax.dot_general(oh1, kp3d_ref[...],
                           (((2,), (1,)), ((0,), (0,))),
                           precision=_HI)

    costs = []
    for g in range(_G):
        gt_g = gt_ref[:, g, :][:, None, :]
        v_g = valid_ref[:, g, :][:, None, :]
        costs.append(jnp.sum(jnp.abs((selk - gt_g) * v_g), axis=2))
    cost = jnp.stack(costs, axis=1)

    used = jnp.zeros((_B, _NS), jnp.bool_)
    fq_cols, sc_cols, lb_cols = [], [], []
    for g in range(_G):
        c = jnp.where(used, jnp.inf, cost[:, g, :])
        m = jnp.min(c, axis=1, keepdims=True)
        idx_g = jnp.min(jnp.where(c == m, iota100, _NS), axis=1,
                        keepdims=True)
        sel = iota100 == idx_g
        used = jnp.logical_or(used, sel)
        sc_cols.append(jnp.sum(sel.astype(f32) * vals, axis=1,
                               keepdims=True))
        lb_cols.append(jnp.sum(jnp.where(sel, lab, 0), axis=1,
                               keepdims=True))
        fq_cols.append(jnp.sum(jnp.where(sel, tb, 0), axis=1, keepdims=True))

    fq = jnp.concatenate(fq_cols, axis=1)
    scores_ref[...] = jnp.concatenate(sc_cols, axis=1)
    labels_ref[...] = jnp.concatenate(lb_cols, axis=1)
    fqflat_ref[...] = fq

    oh2 = (lax.broadcasted_iota(i32, (_B, _G, _Q), 2)
           == fq[:, :, None]).astype(f32)
    sel8 = lax.dot_general(oh2, feat_ref[...],
                           (((2,), (1,)), ((0,), (0,))),
                           precision=_HI)
    t8 = lax.dot_general(sel8, m_ref[...],
                         (((2,), (0,)), ((), ())),
                         precision=_HI)
    w = ts_ref[:, 1:2]
    h = ts_ref[:, 0:1]
    d = w * d3_ref[0:1, :] + h * d3_ref[1:2, :] + d3_ref[2:3, :]
    big_ref[...] = t8 * d[:, None, :]


def _gather_rows_kernel(idx_ref, rows_ref, out_ref):
    b = pl.program_id(0) // _G
    x = rows_ref[0]
    mask = lax.broadcasted_iota(jnp.int32, (3, _B, _V), 1) == b
    out_ref[0, :, 0, :] = jnp.sum(jnp.where(mask, x, 0.0), axis=1)


@jax.jit
def kernel(pred_logits, pred_boxes, pred_keypoints, pred_smpl_fullpose,
           pred_smpl_beta, pred_smpl_expr, pred_smpl_cam, pred_smpl_kp3d,
           pred_smpl_verts, target_sizes, joint_cam, joint_valid,
           body_bbox_center, body_bbox_size):
    del body_bbox_center, body_bbox_size
    f32 = jnp.float32
    prob = jax.nn.sigmoid(pred_logits).reshape(_B, _Q * _C)
    kp3d_flat = pred_smpl_kp3d.reshape(_B, _Q, _JF)
    feat = jnp.concatenate(
        [pred_boxes, pred_keypoints, pred_smpl_fullpose, pred_smpl_beta,
         pred_smpl_expr, pred_smpl_cam, kp3d_flat], axis=-1)
    gt = joint_cam.reshape(_B, _G, _JF)
    validx = jnp.broadcast_to(joint_valid, (_B, _G, _J, 3)).reshape(
        _B, _G, _JF)

    scores, labels, fq8, big = pl.pallas_call(
        _select_kernel,
        out_shape=[
            jax.ShapeDtypeStruct((_B, _G), f32),
            jax.ShapeDtypeStruct((_B, _G), jnp.int32),
            jax.ShapeDtypeStruct((_B, _G), jnp.int32),
            jax.ShapeDtypeStruct((_B, _G, _F), f32),
        ],
    )(prob, feat, kp3d_flat, gt, validx, target_sizes,
      jnp.asarray(_M_CONST), jnp.asarray(_D3_CONST))

    verts_t = jnp.transpose(pred_smpl_verts, (1, 3, 0, 2))
    verts_sel = pl.pallas_call(
        _gather_rows_kernel,
        grid_spec=pltpu.PrefetchScalarGridSpec(
            num_scalar_prefetch=1,
            grid=(_B * _G,),
            in_specs=[pl.BlockSpec((1, 3, _B, _V),
                                   lambda i, idx: (idx[i], 0, 0, 0))],
            out_specs=pl.BlockSpec((1, 3, 1, _V),
                                   lambda i, idx: (i, 0, 0, 0)),
        ),
        out_shape=jax.ShapeDtypeStruct((_B * _G, 3, 1, _V), f32),
    )(fq8.reshape(_B * _G), verts_t)

    boxes_sc = big[..., _OFF_BOX:_OFF_BOX + 4]
    kr = big[..., _OFF_KP:_OFF_KP + 51]
    pose = big[..., _OFF_POSE:_OFF_POSE + 159]
    beta = big[..., _OFF_BETA:_OFF_BETA + 10]
    expr = big[..., _OFF_EXPR:_OFF_EXPR + 10]
    cam = big[..., _OFF_CAM:_OFF_CAM + 3]
    kp3d = big[..., _OFF_KP3D:].reshape(_B, _G, _J, 3)
    verts = jnp.transpose(verts_sel.reshape(_B, _G, 3, _V),
                          (0, 1, 3, 2))
    return (scores, labels, boxes_sc, kr, pose, beta, expr, cam, kp3d, verts)
